# baseline (speedup 1.0000x reference)
import functools

import jax
import jax.numpy as jnp
from jax import lax
from jax.experimental import pallas as pl
from jax.experimental.pallas import tpu as pltpu
from jax.experimental.pallas import tpu_sc as plsc

N = 10000
E = 320000
D = 128

NC = 2
NS = 16
NW = NC * NS
L = 16

NP = 10240
EP = 327680
EPW = EP // NW
B = 128
NB = EPW // B
RPT = NP // NS

_mesh = plsc.VectorSubcoreMesh(core_axis_name="c", subcore_axis_name="s")


@functools.partial(
    pl.kernel,
    out_type=jax.ShapeDtypeStruct((NC, NP, D), jnp.float32),
    mesh=_mesh,
    scratch_types=[
        pltpu.VMEM((NB // 2, B), jnp.int32),
        pltpu.VMEM((NB // 2, B), jnp.int32),
        pltpu.VMEM((B, D), jnp.float32),
        pltpu.VMEM((B, D), jnp.float32),
        pltpu.VMEM_SHARED((NP, D), jnp.float32),
        pltpu.SemaphoreType.DMA,
        pltpu.SemaphoreType.DMA,
    ],
    compiler_params=pltpu.CompilerParams(needs_layout_passes=False),
)
def _segment_sum(src_hbm, tgt_hbm, x_hbm, zeros_hbm, out_hbm,
                 idx_s, idx_t, rows0, rows1, agg_sh, sem0, sem1):
    c = lax.axis_index("c")
    s = lax.axis_index("s")
    wid = c * NS + s
    rows = (rows0, rows1)
    sems = (sem0, sem1)
    NBH = NB // 2

    pltpu.sync_copy(zeros_hbm, agg_sh.at[pl.ds(s * RPT, RPT)])
    plsc.subcore_barrier()

    for half in range(2):
        pltpu.sync_copy(src_hbm.at[wid, pl.ds(half * NBH, NBH)], idx_s)
        pltpu.sync_copy(tgt_hbm.at[wid, pl.ds(half * NBH, NBH)], idx_t)

        for b in range(2):
            pltpu.async_copy(x_hbm.at[idx_s.at[b]], rows[b], sems[b])

        def it_body(it, carry):
            for b in range(2):
                i = it * 2 + b
                pltpu.make_async_copy(x_hbm.at[pl.ds(0, B)], rows[b], sems[b]).wait()
                pltpu.sync_copy(rows[b], agg_sh.at[idx_t.at[i]], add=True)
                inext = jnp.minimum(i + 2, NBH - 1)
                pltpu.async_copy(x_hbm.at[idx_s.at[inext]], rows[b], sems[b])
            return carry

        lax.fori_loop(0, NBH // 2, it_body, 0)
        for b in range(2):
            pltpu.make_async_copy(x_hbm.at[pl.ds(0, B)], rows[b], sems[b]).wait()
    plsc.subcore_barrier()

    pltpu.sync_copy(agg_sh.at[pl.ds(s * RPT, RPT)],
                    out_hbm.at[c, pl.ds(s * RPT, RPT)])


_RB = 1024


def _encoder_body(agg_ref, x_ref, wn_ref, ws_ref, b_ref, o_ref):
    agg = agg_ref[0] + agg_ref[1]
    acc = jnp.dot(agg, wn_ref[...], preferred_element_type=jnp.float32)
    acc += jnp.dot(x_ref[...], ws_ref[...], preferred_element_type=jnp.float32)
    acc += b_ref[...]
    o_ref[...] = jnp.maximum(acc, 0.0).astype(jnp.bfloat16)


_encoder = pl.pallas_call(
    _encoder_body,
    grid=(NP // _RB,),
    in_specs=[
        pl.BlockSpec((NC, _RB, D), lambda i: (0, i, 0)),
        pl.BlockSpec((_RB, D), lambda i: (i, 0)),
        pl.BlockSpec((D, D), lambda i: (0, 0)),
        pl.BlockSpec((D, D), lambda i: (0, 0)),
        pl.BlockSpec((1, D), lambda i: (0, 0)),
    ],
    out_specs=pl.BlockSpec((_RB, D), lambda i: (i, 0)),
    out_shape=jax.ShapeDtypeStruct((NP, D), jnp.bfloat16),
)


@functools.partial(
    pl.kernel,
    out_type=jax.ShapeDtypeStruct((EP,), jnp.float32),
    mesh=_mesh,
    scratch_types=[
        pltpu.VMEM((NB, B), jnp.int32),
        pltpu.VMEM((NB, B), jnp.int32),
        pltpu.VMEM((B, D // 2), jnp.int32),
        pltpu.VMEM((B, D // 2), jnp.int32),
        pltpu.VMEM((B, D // 2), jnp.int32),
        pltpu.VMEM((B, D // 2), jnp.int32),
        pltpu.VMEM((B,), jnp.float32),
        pltpu.SemaphoreType.DMA,
        pltpu.SemaphoreType.DMA,
    ],
    compiler_params=pltpu.CompilerParams(
        needs_layout_passes=False, use_tc_tiling_on_sc=False),
)
def _edge_dots(src_hbm, tgt_hbm, h_hbm, out_hbm,
               idx_s, idx_t, rs0, rs1, rt0, rt1, out_v, sem0, sem1):
    c = lax.axis_index("c")
    s = lax.axis_index("s")
    wid = c * NS + s
    rows_s = (rs0, rs1)
    rows_t = (rt0, rt1)
    sems = (sem0, sem1)

    pltpu.sync_copy(src_hbm.at[wid], idx_s)
    pltpu.sync_copy(tgt_hbm.at[wid], idx_t)

    ebase = wid * EPW

    for b in range(2):
        pltpu.async_copy(h_hbm.at[idx_s.at[b]], rows_s[b], sems[b])
        pltpu.async_copy(h_hbm.at[idx_t.at[b]], rows_t[b], sems[b])

    def it_body(it, carry):
        for b in range(2):
            i = it * 2 + b
            pltpu.make_async_copy(h_hbm.at[pl.ds(0, B)], rows_s[b], sems[b]).wait()
            pltpu.make_async_copy(h_hbm.at[pl.ds(0, B)], rows_t[b], sems[b]).wait()
            rs, rt = rows_s[b], rows_t[b]

            @plsc.parallel_loop(0, B // L, step=1, unroll=4)
            def g_body(g):
                res = jnp.zeros((L,), jnp.float32)
                for j in range(L):
                    e = g * L + j
                    acc = None
                    for k in range(D // 32):
                        vs = plsc.bitcast(rs[e, pl.ds(k * L, L)], jnp.bfloat16)
                        vt = plsc.bitcast(rt[e, pl.ds(k * L, L)], jnp.bfloat16)
                        pa, pb = plsc.unpack(vs * vt,
                                             format=plsc.PackFormat.INTERLEAVED)
                        p = pa + pb
                        acc = p if acc is None else acc + p
                    tot = jnp.sum(acc)
                    onehot = (lax.iota(jnp.int32, L) == j).astype(jnp.float32)
                    res = res + tot * onehot
                out_v[pl.ds(g * L, L)] = res

            pltpu.sync_copy(out_v, out_hbm.at[pl.ds(ebase + i * B, B)])
            inext = jnp.minimum(i + 2, NB - 1)
            pltpu.async_copy(h_hbm.at[idx_s.at[inext]], rows_s[b], sems[b])
            pltpu.async_copy(h_hbm.at[idx_t.at[inext]], rows_t[b], sems[b])
        return carry

    lax.fori_loop(0, NB // 2, it_body, 0)
    for b in range(2):
        pltpu.make_async_copy(h_hbm.at[pl.ds(0, B)], rows_s[b], sems[b]).wait()
        pltpu.make_async_copy(h_hbm.at[pl.ds(0, B)], rows_t[b], sems[b]).wait()


def kernel(x, edge_index, W_neigh, W_self, b):
    src = edge_index[0]
    tgt = edge_index[1]
    npad = EP - E
    pad_ids = jnp.arange(npad, dtype=jnp.int32)
    src_p = jnp.concatenate([src, pad_ids % N]).reshape(NW, NB, B)
    tgt_p = jnp.concatenate([tgt, N + (pad_ids % (NP - N))]).reshape(NW, NB, B)
    xp = jnp.pad(x, ((0, NP - N), (0, 0)))
    zeros = jnp.zeros((RPT, D), jnp.float32)

    agg2 = _segment_sum(src_p, tgt_p, xp, zeros)
    h = _encoder(agg2, xp, W_neigh, W_self, b.reshape(1, D))
    h32 = lax.bitcast_convert_type(h.reshape(NP, D // 2, 2), jnp.int32)
    scores = _edge_dots(src_p, tgt_p, h32)
    return scores.reshape(EP)[:E]
